# baseline (speedup 1.0000x reference)
import functools

import jax
import jax.numpy as jnp
from jax import lax
from jax.experimental import pallas as pl
from jax.experimental.pallas import tpu as pltpu
from jax.experimental.pallas import tpu_sc as plsc

K = 16
RB = 392
NSUB = 32
R_CH = 8
IW = 2
LANES = 16


def _normalize_body(x_ref, xn_ref, xnt_ref):
    v = x_ref[0]
    sq = jnp.sum(v * v, axis=0, keepdims=True)
    n = jnp.sqrt(sq)
    xn = v / jnp.maximum(n, 1e-12)
    xn_ref[0] = xn
    xnt_ref[0] = xn.T


def _dist_body(xnt_ref, xn_ref, nd_ref):
    lhs = xnt_ref[0]
    rhs = xn_ref[0]
    sqi = jnp.sum(lhs * lhs, axis=1, keepdims=True)
    sqj = jnp.sum(rhs * rhs, axis=0, keepdims=True)
    g = jax.lax.dot_general(lhs, rhs, (((1,), (0,)), ((), ())),
                            preferred_element_type=jnp.float32)
    d = (sqi + (-2.0 * g)) + sqj
    nd_ref[0] = -d


def _topk_sc_body(nd_hbm, out_hbm, buf, cia, cib, cic, cid, ob, sems, *,
                  n_points, rows_per):
    wid = lax.axis_index("s") * 2 + lax.axis_index("c")
    row0 = wid * rows_per
    n_groups = rows_per // R_CH
    n_chunks = n_points // LANES
    iota16 = lax.iota(jnp.int32, LANES)
    neg_inf = jnp.full((LANES,), -jnp.inf, jnp.float32)
    zeros_i = jnp.zeros((LANES,), jnp.int32)
    sixteen = jnp.full((LANES,), LANES, jnp.int32)

    pltpu.make_async_copy(
        nd_hbm.at[pl.ds(row0, R_CH)], buf.at[pl.ds(0, R_CH)],
        sems.at[0]).start()

    def group_body(g, _):
        cur = lax.rem(g, 2)
        base = cur * R_CH
        pltpu.make_async_copy(
            nd_hbm.at[pl.ds(row0 + g * R_CH, R_CH)],
            buf.at[pl.ds(base, R_CH)], sems.at[cur]).wait()

        @pl.when(g + 1 < n_groups)
        def _prefetch():
            pltpu.make_async_copy(
                nd_hbm.at[pl.ds(row0 + (g + 1) * R_CH, R_CH)],
                buf.at[pl.ds((1 - cur) * R_CH, R_CH)],
                sems.at[1 - cur]).start()

        cis = [cia, cib, cic, cid]

        def quad_body(r4, _):
            brs = [base + IW * r4 + w for w in range(IW)]

            def p1(j, accs):
                return tuple(
                    jnp.maximum(accs[w], buf[brs[w], pl.ds(j * LANES, LANES)])
                    for w in range(IW))
            accs = lax.fori_loop(0, n_chunks, p1, (neg_inf,) * IW, unroll=14)
            tvs = [jnp.full((LANES,), lax.sort(a)[0], jnp.float32)
                   for a in accs]

            def p2(j, carry):
                idxv = carry[IW]
                new_cnts = []
                for w in range(IW):
                    m = buf[brs[w], pl.ds(j * LANES, LANES)] >= tvs[w]
                    plsc.store_compressed(cis[w].at[pl.ds(carry[w], LANES)],
                                          idxv, mask=m)
                    new_cnts.append(
                        carry[w] + plsc.all_reduce_population_count(m)[0])
                return tuple(new_cnts) + (idxv + sixteen,)
            out_carry = lax.fori_loop(
                0, n_chunks, p2,
                (jnp.int32(0),) * IW + (iota16,), unroll=7)
            cnts = out_carry[:IW]
            for w in range(IW):
                cis[w][pl.ds(cnts[w], LANES)] = zeros_i

            def merge_rows(br, ci, cnt):
                br_splat = jnp.full((LANES,), br, jnp.int32)

                def p3(j, carry):
                    sv, si = carry
                    cidx_raw = ci[pl.ds(j * LANES, LANES)]
                    valid = (jnp.full((LANES,), j * LANES, jnp.int32)
                             + iota16 < cnt)
                    cvals = jnp.where(
                        valid, plsc.load_gather(buf, [br_splat, cidx_raw]),
                        neg_inf)
                    ck, cidx = plsc.sort_key_val(cvals, cidx_raw,
                                                 descending=True)
                    sel = sv >= ck
                    nv = jnp.where(sel, sv, ck)
                    ni = jnp.where(sel, si, cidx)
                    return tuple(plsc.sort_key_val(nv, ni, descending=False))
                nch = (cnt + 15) >> 4
                sv, si = lax.fori_loop(0, nch, p3, (neg_inf, zeros_i))
                return lax.rev(si, (0,))
            for w in range(IW):
                ob[g * R_CH + IW * r4 + w, :] = merge_rows(
                    brs[w], cis[w], cnts[w])
            return 0
        lax.fori_loop(0, R_CH // IW, quad_body, 0)
        return 0
    lax.fori_loop(0, n_groups, group_body, 0)
    pltpu.sync_copy(ob, out_hbm.at[pl.ds(row0, rows_per)])


def kernel(x):
    B, C, H, W = x.shape
    N = H * W
    xf = x.reshape(B, C, N)

    xn, xnt = pl.pallas_call(
        _normalize_body,
        grid=(B,),
        in_specs=[pl.BlockSpec((1, C, N), lambda b: (b, 0, 0))],
        out_specs=[
            pl.BlockSpec((1, C, N), lambda b: (b, 0, 0)),
            pl.BlockSpec((1, N, C), lambda b: (b, 0, 0)),
        ],
        out_shape=[
            jax.ShapeDtypeStruct((B, C, N), jnp.float32),
            jax.ShapeDtypeStruct((B, N, C), jnp.float32),
        ],
    )(xf)

    nd = pl.pallas_call(
        _dist_body,
        grid=(B, N // RB),
        in_specs=[
            pl.BlockSpec((1, RB, C), lambda b, r: (b, r, 0)),
            pl.BlockSpec((1, C, N), lambda b, r: (b, 0, 0)),
        ],
        out_specs=pl.BlockSpec((1, RB, N), lambda b, r: (b, r, 0)),
        out_shape=jax.ShapeDtypeStruct((B, N, N), jnp.float32),
    )(xnt, xn)

    rows = B * N
    rows_per = rows // NSUB
    nd_flat = nd.reshape(rows, N)

    topk = pl.kernel(
        functools.partial(_topk_sc_body, n_points=N, rows_per=rows_per),
        out_type=jax.ShapeDtypeStruct((rows, K), jnp.int32),
        mesh=plsc.VectorSubcoreMesh(core_axis_name="c", subcore_axis_name="s",
                                    num_cores=2, num_subcores=16),
        compiler_params=pltpu.CompilerParams(needs_layout_passes=False),
        scratch_types=[
            pltpu.VMEM((2 * R_CH, N), jnp.float32),
            pltpu.VMEM((N + LANES,), jnp.int32),
            pltpu.VMEM((N + LANES,), jnp.int32),
            pltpu.VMEM((N + LANES,), jnp.int32),
            pltpu.VMEM((N + LANES,), jnp.int32),
            pltpu.VMEM((rows_per, K), jnp.int32),
            pltpu.SemaphoreType.DMA((2,)),
        ],
    )
    nn_idx = topk(nd_flat).reshape(B, N, K)

    center_idx = jnp.broadcast_to(
        jnp.arange(N, dtype=jnp.int32)[None, :, None], (B, N, K))
    return jnp.stack((nn_idx, center_idx), axis=0)


# speedup vs baseline: 1.4398x; 1.4398x over previous
import functools

import jax
import jax.numpy as jnp
from jax import lax
from jax.experimental import pallas as pl
from jax.experimental.pallas import tpu as pltpu
from jax.experimental.pallas import tpu_sc as plsc

K = 16
RB = 392
NSUB = 32
R_CH = 8
IW = 2
LANES = 16


def _normalize_body(x_ref, xn_ref, xnt_ref):
    v = x_ref[0]
    sq = jnp.sum(v * v, axis=0, keepdims=True)
    n = jnp.sqrt(sq)
    xn = v / jnp.maximum(n, 1e-12)
    xn_ref[0] = xn
    xnt_ref[0] = xn.T


def _dist_body(xnt_ref, xn_ref, nd_ref):
    lhs = xnt_ref[0]
    rhs = xn_ref[0]
    sqi = jnp.sum(lhs * lhs, axis=1, keepdims=True)
    sqj = jnp.sum(rhs * rhs, axis=0, keepdims=True)
    g = jax.lax.dot_general(lhs, rhs, (((1,), (0,)), ((), ())),
                            preferred_element_type=jnp.float32)
    d = (sqi + (-2.0 * g)) + sqj
    nd_ref[0] = -d


def _topk_sc_body(nd_hbm, out_hbm, buf, cia, cib, cic, cid, ob, sems, *,
                  n_points, rows_per):
    wid = lax.axis_index("s") * 2 + lax.axis_index("c")
    row0 = wid * rows_per
    n_groups = rows_per // R_CH
    n_chunks = n_points // LANES
    iota16 = lax.iota(jnp.int32, LANES)
    neg_inf = jnp.full((LANES,), -jnp.inf, jnp.float32)
    zeros_i = jnp.zeros((LANES,), jnp.int32)
    sixteen = jnp.full((LANES,), LANES, jnp.int32)

    pltpu.make_async_copy(
        nd_hbm.at[pl.ds(row0, R_CH)], buf.at[pl.ds(0, R_CH)],
        sems.at[0]).start()

    def group_body(g, _):
        cur = lax.rem(g, 2)
        base = cur * R_CH
        pltpu.make_async_copy(
            nd_hbm.at[pl.ds(row0 + g * R_CH, R_CH)],
            buf.at[pl.ds(base, R_CH)], sems.at[cur]).wait()

        @pl.when(g + 1 < n_groups)
        def _prefetch():
            pltpu.make_async_copy(
                nd_hbm.at[pl.ds(row0 + (g + 1) * R_CH, R_CH)],
                buf.at[pl.ds((1 - cur) * R_CH, R_CH)],
                sems.at[1 - cur]).start()

        cis = [cia, cib, cic, cid]

        def quad_body(r4, _):
            brs = [base + IW * r4 + w for w in range(IW)]

            def p1(j, accs):
                return tuple(
                    jnp.maximum(accs[w], buf[brs[w], pl.ds(j * LANES, LANES)])
                    for w in range(IW))
            accs = lax.fori_loop(0, n_chunks, p1, (neg_inf,) * IW, unroll=14)
            tvs = [jnp.full((LANES,), lax.sort(a)[0], jnp.float32)
                   for a in accs]

            def p2(j, carry):
                idxv = carry[IW]
                ms = [buf[brs[w], pl.ds(j * LANES, LANES)] >= tvs[w]
                      for w in range(IW)]
                for w in range(IW):
                    plsc.store_compressed(cis[w].at[pl.ds(carry[w], LANES)],
                                          idxv, mask=ms[w])
                new_cnts = [
                    carry[w] + plsc.all_reduce_population_count(ms[w])[0]
                    for w in range(IW)]
                return tuple(new_cnts) + (idxv + sixteen,)
            out_carry = lax.fori_loop(
                0, n_chunks, p2,
                (jnp.int32(0),) * IW + (iota16,), unroll=7)
            cnts = out_carry[:IW]
            for w in range(IW):
                cis[w][pl.ds(cnts[w], LANES)] = zeros_i

            def merge_rows(br, ci, cnt):
                br_splat = jnp.full((LANES,), br, jnp.int32)

                def p3(j, carry):
                    sv, si = carry
                    cidx_raw = ci[pl.ds(j * LANES, LANES)]
                    valid = (jnp.full((LANES,), j * LANES, jnp.int32)
                             + iota16 < cnt)
                    cvals = jnp.where(
                        valid, plsc.load_gather(buf, [br_splat, cidx_raw]),
                        neg_inf)
                    ck, cidx = plsc.sort_key_val(cvals, cidx_raw,
                                                 descending=True)
                    sel = sv >= ck
                    nv = jnp.where(sel, sv, ck)
                    ni = jnp.where(sel, si, cidx)
                    return tuple(plsc.sort_key_val(nv, ni, descending=False))
                nch = (cnt + 15) >> 4
                sv, si = lax.fori_loop(0, nch, p3, (neg_inf, zeros_i))
                return lax.rev(si, (0,))
            for w in range(IW):
                ob[g * R_CH + IW * r4 + w, :] = merge_rows(
                    brs[w], cis[w], cnts[w])
            return 0
        lax.fori_loop(0, R_CH // IW, quad_body, 0)
        return 0
    lax.fori_loop(0, n_groups, group_body, 0)
    pltpu.sync_copy(ob, out_hbm.at[pl.ds(row0, rows_per)])


def kernel(x):
    B, C, H, W = x.shape
    N = H * W
    xf = x.reshape(B, C, N)

    xn, xnt = pl.pallas_call(
        _normalize_body,
        grid=(B,),
        in_specs=[pl.BlockSpec((1, C, N), lambda b: (b, 0, 0))],
        out_specs=[
            pl.BlockSpec((1, C, N), lambda b: (b, 0, 0)),
            pl.BlockSpec((1, N, C), lambda b: (b, 0, 0)),
        ],
        out_shape=[
            jax.ShapeDtypeStruct((B, C, N), jnp.float32),
            jax.ShapeDtypeStruct((B, N, C), jnp.float32),
        ],
    )(xf)

    nd = pl.pallas_call(
        _dist_body,
        grid=(B, N // RB),
        in_specs=[
            pl.BlockSpec((1, RB, C), lambda b, r: (b, r, 0)),
            pl.BlockSpec((1, C, N), lambda b, r: (b, 0, 0)),
        ],
        out_specs=pl.BlockSpec((1, RB, N), lambda b, r: (b, r, 0)),
        out_shape=jax.ShapeDtypeStruct((B, N, N), jnp.float32),
    )(xnt, xn)

    rows = B * N
    rows_per = rows // NSUB
    nd_flat = nd.reshape(rows, N)

    topk = pl.kernel(
        functools.partial(_topk_sc_body, n_points=N, rows_per=rows_per),
        out_type=jax.ShapeDtypeStruct((rows, K), jnp.int32),
        mesh=plsc.VectorSubcoreMesh(core_axis_name="c", subcore_axis_name="s",
                                    num_cores=2, num_subcores=16),
        compiler_params=pltpu.CompilerParams(needs_layout_passes=False),
        scratch_types=[
            pltpu.VMEM((2 * R_CH, N), jnp.float32),
            pltpu.VMEM((N + LANES,), jnp.int32),
            pltpu.VMEM((N + LANES,), jnp.int32),
            pltpu.VMEM((N + LANES,), jnp.int32),
            pltpu.VMEM((N + LANES,), jnp.int32),
            pltpu.VMEM((rows_per, K), jnp.int32),
            pltpu.SemaphoreType.DMA((2,)),
        ],
    )
    nn_idx = topk(nd_flat).reshape(B, N, K)

    center_idx = jnp.broadcast_to(
        jnp.arange(N, dtype=jnp.int32)[None, :, None], (B, N, K))
    return jnp.stack((nn_idx, center_idx), axis=0)


# speedup vs baseline: 2.0483x; 1.4227x over previous
import functools

import jax
import jax.numpy as jnp
from jax import lax
from jax.experimental import pallas as pl
from jax.experimental.pallas import tpu as pltpu
from jax.experimental.pallas import tpu_sc as plsc

K = 16
RB = 392
NSUB = 32
R_CH = 8
IW = 4
LANES = 16


def _normalize_body(x_ref, xn_ref, xnt_ref):
    v = x_ref[0]
    sq = jnp.sum(v * v, axis=0, keepdims=True)
    n = jnp.sqrt(sq)
    xn = v / jnp.maximum(n, 1e-12)
    xn_ref[0] = xn
    xnt_ref[0] = xn.T


def _dist_body(xnt_ref, xn_ref, nd_ref):
    lhs = xnt_ref[0]
    rhs = xn_ref[0]
    sqi = jnp.sum(lhs * lhs, axis=1, keepdims=True)
    sqj = jnp.sum(rhs * rhs, axis=0, keepdims=True)
    g = jax.lax.dot_general(lhs, rhs, (((1,), (0,)), ((), ())),
                            preferred_element_type=jnp.float32)
    d = (sqi + (-2.0 * g)) + sqj
    nd_ref[0] = -d


def _topk_sc_body(nd_hbm, out_hbm, buf, cia, cib, cic, cid, ob, sems, *,
                  n_points, rows_per):
    wid = lax.axis_index("s") * 2 + lax.axis_index("c")
    row0 = wid * rows_per
    n_groups = rows_per // R_CH
    n_chunks = n_points // LANES
    iota16 = lax.iota(jnp.int32, LANES)
    neg_inf = jnp.full((LANES,), -jnp.inf, jnp.float32)
    zeros_i = jnp.zeros((LANES,), jnp.int32)
    sixteen = jnp.full((LANES,), LANES, jnp.int32)

    pltpu.make_async_copy(
        nd_hbm.at[pl.ds(row0, R_CH)], buf.at[pl.ds(0, R_CH)],
        sems.at[0]).start()

    def group_body(g, _):
        cur = lax.rem(g, 2)
        base = cur * R_CH
        pltpu.make_async_copy(
            nd_hbm.at[pl.ds(row0 + g * R_CH, R_CH)],
            buf.at[pl.ds(base, R_CH)], sems.at[cur]).wait()

        @pl.when(g + 1 < n_groups)
        def _prefetch():
            pltpu.make_async_copy(
                nd_hbm.at[pl.ds(row0 + (g + 1) * R_CH, R_CH)],
                buf.at[pl.ds((1 - cur) * R_CH, R_CH)],
                sems.at[1 - cur]).start()

        cis = [cia, cib, cic, cid]

        def quad_body(r4, _):
            brs = [base + IW * r4 + w for w in range(IW)]

            def p1(j, accs):
                return tuple(
                    jnp.maximum(accs[w], buf[brs[w], pl.ds(j * LANES, LANES)])
                    for w in range(IW))
            accs = lax.fori_loop(0, n_chunks, p1, (neg_inf,) * IW, unroll=14)
            tvs = [jnp.full((LANES,), lax.sort(a)[0], jnp.float32)
                   for a in accs]

            def p2(j, carry):
                idxv = carry[IW]
                ms = [buf[brs[w], pl.ds(j * LANES, LANES)] >= tvs[w]
                      for w in range(IW)]
                for w in range(IW):
                    plsc.store_compressed(cis[w].at[pl.ds(carry[w], LANES)],
                                          idxv, mask=ms[w])
                new_cnts = [
                    carry[w] + plsc.all_reduce_population_count(ms[w])[0]
                    for w in range(IW)]
                return tuple(new_cnts) + (idxv + sixteen,)
            out_carry = lax.fori_loop(
                0, n_chunks, p2,
                (jnp.int32(0),) * IW + (iota16,), unroll=7)
            cnts = out_carry[:IW]
            for w in range(IW):
                cis[w][pl.ds(cnts[w], LANES)] = zeros_i

            def merge_rows(br, ci, cnt):
                br_splat = jnp.full((LANES,), br, jnp.int32)

                def p3(j, carry):
                    sv, si = carry
                    cidx_raw = ci[pl.ds(j * LANES, LANES)]
                    valid = (jnp.full((LANES,), j * LANES, jnp.int32)
                             + iota16 < cnt)
                    cvals = jnp.where(
                        valid, plsc.load_gather(buf, [br_splat, cidx_raw]),
                        neg_inf)
                    ck, cidx = plsc.sort_key_val(cvals, cidx_raw,
                                                 descending=True)
                    sel = sv >= ck
                    nv = jnp.where(sel, sv, ck)
                    ni = jnp.where(sel, si, cidx)
                    return tuple(plsc.sort_key_val(nv, ni, descending=False))
                nch = (cnt + 15) >> 4
                sv, si = lax.fori_loop(0, nch, p3, (neg_inf, zeros_i))
                return lax.rev(si, (0,))
            for w in range(IW):
                ob[g * R_CH + IW * r4 + w, :] = merge_rows(
                    brs[w], cis[w], cnts[w])
            return 0
        lax.fori_loop(0, R_CH // IW, quad_body, 0)
        return 0
    lax.fori_loop(0, n_groups, group_body, 0)
    pltpu.sync_copy(ob, out_hbm.at[pl.ds(row0, rows_per)])


def kernel(x):
    B, C, H, W = x.shape
    N = H * W
    xf = x.reshape(B, C, N)

    xn, xnt = pl.pallas_call(
        _normalize_body,
        grid=(B,),
        in_specs=[pl.BlockSpec((1, C, N), lambda b: (b, 0, 0))],
        out_specs=[
            pl.BlockSpec((1, C, N), lambda b: (b, 0, 0)),
            pl.BlockSpec((1, N, C), lambda b: (b, 0, 0)),
        ],
        out_shape=[
            jax.ShapeDtypeStruct((B, C, N), jnp.float32),
            jax.ShapeDtypeStruct((B, N, C), jnp.float32),
        ],
    )(xf)

    nd = pl.pallas_call(
        _dist_body,
        grid=(B, N // RB),
        in_specs=[
            pl.BlockSpec((1, RB, C), lambda b, r: (b, r, 0)),
            pl.BlockSpec((1, C, N), lambda b, r: (b, 0, 0)),
        ],
        out_specs=pl.BlockSpec((1, RB, N), lambda b, r: (b, r, 0)),
        out_shape=jax.ShapeDtypeStruct((B, N, N), jnp.float32),
    )(xnt, xn)

    rows = B * N
    rows_per = rows // NSUB
    nd_flat = nd.reshape(rows, N)

    topk = pl.kernel(
        functools.partial(_topk_sc_body, n_points=N, rows_per=rows_per),
        out_type=jax.ShapeDtypeStruct((rows, K), jnp.int32),
        mesh=plsc.VectorSubcoreMesh(core_axis_name="c", subcore_axis_name="s",
                                    num_cores=2, num_subcores=16),
        compiler_params=pltpu.CompilerParams(needs_layout_passes=False),
        scratch_types=[
            pltpu.VMEM((2 * R_CH, N), jnp.float32),
            pltpu.VMEM((N + LANES,), jnp.int32),
            pltpu.VMEM((N + LANES,), jnp.int32),
            pltpu.VMEM((N + LANES,), jnp.int32),
            pltpu.VMEM((N + LANES,), jnp.int32),
            pltpu.VMEM((rows_per, K), jnp.int32),
            pltpu.SemaphoreType.DMA((2,)),
        ],
    )
    nn_idx = topk(nd_flat).reshape(B, N, K)

    center_idx = jnp.broadcast_to(
        jnp.arange(N, dtype=jnp.int32)[None, :, None], (B, N, K))
    return jnp.stack((nn_idx, center_idx), axis=0)
